# Initial kernel scaffold; baseline (speedup 1.0000x reference)
import jax
import jax.numpy as jnp
from jax.experimental import pallas as pl


def kernel(data, org_edge_index, edge_index_set0, W1, att_src1, att_dst1, bias1, W2, att_src2, att_dst2, bias2, Wfc, bfc, bn_gamma, bn_beta):
    raise NotImplementedError("write your pallas kernel here")


# speedup vs baseline: 60.2697x; 60.2697x over previous
import functools

import jax
import jax.numpy as jnp
import numpy as np
from jax import lax
from jax.experimental import pallas as pl
from jax.experimental.pallas import tpu as pltpu
from jax.experimental.pallas import tpu_sc as plsc

NN = 10000
BB = 4
EE = 160000
NT = 16
NC = 2
EPT = EE // NT
CH = 80
NCHUNK = EPT // CH
ROWS = 2000
ACCW = 144
LSW = 16


def _leaky(v):
    return jnp.where(v >= 0.0, v, 0.2 * v)


def _pre_body(x_ref, w_ref, a_ref, xw_ref, lsld_ref):
    xw = jnp.dot(x_ref[...], w_ref[...], preferred_element_type=jnp.float32,
                 precision=lax.Precision.HIGHEST)
    xw_ref[...] = xw
    lsld_ref[...] = jnp.dot(xw, a_ref[...], preferred_element_type=jnp.float32,
                 precision=lax.Precision.HIGHEST)


def _tc_pre(x, W, A):
    din = x.shape[1]
    return pl.pallas_call(
        _pre_body,
        grid=(x.shape[0] // ROWS,),
        in_specs=[
            pl.BlockSpec((ROWS, din), lambda i: (i, 0)),
            pl.BlockSpec((din, 128), lambda i: (0, 0)),
            pl.BlockSpec((128, LSW), lambda i: (0, 0)),
        ],
        out_specs=[
            pl.BlockSpec((ROWS, 128), lambda i: (i, 0)),
            pl.BlockSpec((ROWS, LSW), lambda i: (i, 0)),
        ],
        out_shape=[
            jax.ShapeDtypeStruct((x.shape[0], 128), jnp.float32),
            jax.ShapeDtypeStruct((x.shape[0], LSW), jnp.float32),
        ],
    )(x, W, A)


def _combine(acc, xw, l):
    w0 = jnp.exp(_leaky(l[:, 0:1] + l[:, 2:3]))
    w1 = jnp.exp(_leaky(l[:, 1:2] + l[:, 3:4]))
    den0 = acc[:, 128:129] + w0 + 1e-16
    den1 = acc[:, 129:130] + w1 + 1e-16
    r = acc.shape[0]
    wb = jnp.concatenate(
        [jnp.broadcast_to(w0, (r, 64)), jnp.broadcast_to(w1, (r, 64))], axis=1)
    denb = jnp.concatenate(
        [jnp.broadcast_to(den0, (r, 64)), jnp.broadcast_to(den1, (r, 64))], axis=1)
    return (acc[:, 0:128] + wb * xw) / denb


def _mid_body(acc_ref, xw_ref, lsld_ref, w2_ref, a2_ref, b1_ref, xw2_ref, lsld2_ref):
    x1 = _combine(acc_ref[...], xw_ref[...], lsld_ref[...]) + b1_ref[...]
    xw2 = jnp.dot(x1, w2_ref[...], preferred_element_type=jnp.float32,
                 precision=lax.Precision.HIGHEST)
    xw2_ref[...] = xw2
    lsld2_ref[...] = jnp.dot(xw2, a2_ref[...], preferred_element_type=jnp.float32,
                 precision=lax.Precision.HIGHEST)


def _tc_mid(acc, xw, lsld, W2, A2, b1):
    n = xw.shape[0]
    return pl.pallas_call(
        _mid_body,
        grid=(n // ROWS,),
        in_specs=[
            pl.BlockSpec((ROWS, ACCW), lambda i: (i, 0)),
            pl.BlockSpec((ROWS, 128), lambda i: (i, 0)),
            pl.BlockSpec((ROWS, LSW), lambda i: (i, 0)),
            pl.BlockSpec((128, 128), lambda i: (0, 0)),
            pl.BlockSpec((128, LSW), lambda i: (0, 0)),
            pl.BlockSpec((1, 128), lambda i: (0, 0)),
        ],
        out_specs=[
            pl.BlockSpec((ROWS, 128), lambda i: (i, 0)),
            pl.BlockSpec((ROWS, LSW), lambda i: (i, 0)),
        ],
        out_shape=[
            jax.ShapeDtypeStruct((n, 128), jnp.float32),
            jax.ShapeDtypeStruct((n, LSW), jnp.float32),
        ],
    )(acc, xw, lsld, W2, A2, b1)


def _post_body(acc_ref, xw_ref, lsld_ref, b2_ref, wfc_ref, sc_ref, sh_ref, y_ref):
    x2 = _combine(acc_ref[...], xw_ref[...], lsld_ref[...]) + b2_ref[...]
    t = jnp.dot(x2, wfc_ref[...], preferred_element_type=jnp.float32,
                 precision=lax.Precision.HIGHEST)
    t = t * sc_ref[...] + sh_ref[...]
    y_ref[...] = jnp.maximum(t, 0.0)


def _tc_post(acc, xw, lsld, b2, Wfc, scale, shift):
    n = xw.shape[0]
    return pl.pallas_call(
        _post_body,
        grid=(n // ROWS,),
        in_specs=[
            pl.BlockSpec((ROWS, ACCW), lambda i: (i, 0)),
            pl.BlockSpec((ROWS, 128), lambda i: (i, 0)),
            pl.BlockSpec((ROWS, LSW), lambda i: (i, 0)),
            pl.BlockSpec((1, 128), lambda i: (0, 0)),
            pl.BlockSpec((128, 1), lambda i: (0, 0)),
            pl.BlockSpec((1, 1), lambda i: (0, 0)),
            pl.BlockSpec((1, 1), lambda i: (0, 0)),
        ],
        out_specs=pl.BlockSpec((ROWS, 1), lambda i: (i, 0)),
        out_shape=jax.ShapeDtypeStruct((n, 1), jnp.float32),
    )(acc, xw, lsld, b2, Wfc, scale, shift)


SCH = 25
NSUP = NCHUNK // SCH
RPT = NN // NT


def _sc_body(src_hbm, dst_hbm, lsld_hbm, xw_hbm, out_hbm,
             srcb_v, dstr_v, dsto_v, lsS_v, lsD_v, w_v, g_v, s_v, z_v,
             scope_acc, lsem, gsem):
    c = lax.axis_index("c")
    s = lax.axis_index("s")
    row0 = s * RPT
    iota = lax.iota(jnp.int32, 16)

    def _zero_z(i, _):
        for q in range(ACCW // 16):
            z_v[i, pl.ds(q * 16, 16)] = jnp.zeros((16,), jnp.float32)
        return 0
    lax.fori_loop(0, 25, _zero_z, 0)

    def _zero_acc(k, _):
        pltpu.sync_copy(z_v, scope_acc.at[pl.ds(row0 + k * 25, 25)])
        return 0

    lax.fori_loop(0, RPT // 25, _zero_acc, 0)

    for r in range(BB // NC):
        b = 2 * r + c
        off = b * NN
        plsc.subcore_barrier()

        for sup in range(NSUP):
            pltpu.sync_copy(src_hbm.at[s, pl.ds(sup * SCH, SCH)], srcb_v)
            pltpu.sync_copy(dst_hbm.at[s, pl.ds(sup * SCH, SCH)], dstr_v)

            def _offset(i, _):
                for j in range(CH // 16):
                    sl = pl.ds(j * 16, 16)
                    srcb_v[i, sl] = srcb_v[i, sl] + off
                    dsto_v[i, sl] = dstr_v[i, sl] + off
                return 0

            lax.fori_loop(0, SCH, _offset, 0)

            def _chunk(i, _):
                cp1 = pltpu.make_async_copy(
                    lsld_hbm.at[srcb_v.at[i]], lsS_v, lsem)
                cp1.start()
                cp2 = pltpu.make_async_copy(
                    lsld_hbm.at[dsto_v.at[i]], lsD_v, lsem)
                cp2.start()
                cp3 = pltpu.make_async_copy(
                    xw_hbm.at[srcb_v.at[i]], g_v, gsem)
                cp3.start()
                cp1.wait()
                cp2.wait()

                for jj in range(CH // 16):
                    ridx = iota + jj * 16
                    ls0 = plsc.load_gather(lsS_v, [ridx, iota * 0])
                    ls1 = plsc.load_gather(lsS_v, [ridx, iota * 0 + 1])
                    ld0 = plsc.load_gather(lsD_v, [ridx, iota * 0 + 2])
                    ld1 = plsc.load_gather(lsD_v, [ridx, iota * 0 + 3])
                    w_v[0, pl.ds(jj * 16, 16)] = jnp.exp(_leaky(ls0 + ld0))
                    w_v[1, pl.ds(jj * 16, 16)] = jnp.exp(_leaky(ls1 + ld1))

                cp3.wait()

                def _scale(jj, _2):
                    base = jj * 16
                    w0v = w_v[0, pl.ds(base, 16)]
                    w1v = w_v[1, pl.ds(base, 16)]
                    for l in range(16):
                        j = base + l
                        w0 = w0v[l]
                        w1 = w1v[l]
                        for q in range(4):
                            sl = pl.ds(q * 16, 16)
                            s_v[j, sl] = g_v[j, sl] * w0
                        for q in range(4, 8):
                            sl = pl.ds(q * 16, 16)
                            s_v[j, sl] = g_v[j, sl] * w1
                        dv = jnp.where(iota == 0, w0,
                                       jnp.where(iota == 1, w1, 0.0))
                        s_v[j, pl.ds(128, 16)] = dv
                    return 0

                lax.fori_loop(0, CH // 16, _scale, 0)
                pltpu.sync_copy(s_v, scope_acc.at[dstr_v.at[i]], add=True)
                return 0

            lax.fori_loop(0, SCH, _chunk, 0)

        plsc.subcore_barrier()

        pltpu.sync_copy(scope_acc.at[pl.ds(row0, RPT)],
                        out_hbm.at[b, pl.ds(row0, RPT)])

        if r + 1 < BB // NC:
            lax.fori_loop(0, RPT // 25, _zero_acc, 0)


def _sc_edge(src_t, dst_t, lsld, xw):
    mesh = plsc.VectorSubcoreMesh(core_axis_name="c", subcore_axis_name="s")
    f = pl.kernel(
        _sc_body,
        mesh=mesh,
        compiler_params=pltpu.CompilerParams(
            use_tc_tiling_on_sc=False, needs_layout_passes=False),
        out_type=jax.ShapeDtypeStruct((BB, NN, ACCW), jnp.float32),
        scratch_types=[
            pltpu.VMEM((SCH, CH), jnp.int32),
            pltpu.VMEM((SCH, CH), jnp.int32),
            pltpu.VMEM((SCH, CH), jnp.int32),
            pltpu.VMEM((CH, LSW), jnp.float32),
            pltpu.VMEM((CH, LSW), jnp.float32),
            pltpu.VMEM((2, CH), jnp.float32),
            pltpu.VMEM((CH, 128), jnp.float32),
            pltpu.VMEM((CH, ACCW), jnp.float32),
            pltpu.VMEM((25, ACCW), jnp.float32),
            pltpu.VMEM_SHARED((NN, ACCW), jnp.float32),
            pltpu.SemaphoreType.DMA,
            pltpu.SemaphoreType.DMA,
        ],
    )
    return f(src_t, dst_t, lsld, xw)


def _build_A(a_src, a_dst):
    z = jnp.zeros((64,), jnp.float32)
    top = jnp.stack([a_src[0], z, a_dst[0], z] + [z] * 12, axis=1)
    bot = jnp.stack([z, a_src[1], z, a_dst[1]] + [z] * 12, axis=1)
    return jnp.concatenate([top, bot], axis=0)


def kernel(data, org_edge_index, edge_index_set0, W1, att_src1, att_dst1,
           bias1, W2, att_src2, att_dst2, bias2, Wfc, bfc, bn_gamma, bn_beta):
    x = data.reshape(BB * NN, -1)
    src_t = edge_index_set0[0].astype(jnp.int32).reshape(NT, NCHUNK, CH)
    dst_t = edge_index_set0[1].astype(jnp.int32).reshape(NT, NCHUNK, CH)
    A1 = _build_A(att_src1, att_dst1)
    A2 = _build_A(att_src2, att_dst2)

    xw1, lsld1 = _tc_pre(x, W1, A1)
    acc1 = _sc_edge(src_t, dst_t, lsld1, xw1)
    xw2, lsld2 = _tc_mid(acc1.reshape(BB * NN, ACCW), xw1, lsld1, W2, A2,
                         bias1.reshape(1, 128))
    acc2 = _sc_edge(src_t, dst_t, lsld2, xw2)

    k = 1.0 / np.sqrt(1.0 + 1e-5)
    scale = (bn_gamma * k).reshape(1, 1)
    shift = (bfc * bn_gamma * k + bn_beta).reshape(1, 1)
    y = _tc_post(acc2.reshape(BB * NN, ACCW), xw2, lsld2,
                 bias2.reshape(1, 128), Wfc, scale, shift)
    return y.reshape(BB, 1, NN)


# speedup vs baseline: 156.2062x; 2.5918x over previous
import functools

import jax
import jax.numpy as jnp
import numpy as np
from jax import lax
from jax.experimental import pallas as pl
from jax.experimental.pallas import tpu as pltpu
from jax.experimental.pallas import tpu_sc as plsc

NN = 10000
BB = 4
EE = 160000
NT = 16
NC = 2
EPT = EE // NT
CH = 80
NCHUNK = EPT // CH
ROWS = 2000
ACCW = 144
LSW = 16


def _leaky(v):
    return jnp.where(v >= 0.0, v, 0.2 * v)


def _pre_body(x_ref, w_ref, a_ref, xw_ref, lsld_ref):
    xw = jnp.dot(x_ref[...], w_ref[...], preferred_element_type=jnp.float32,
                 precision=lax.Precision.HIGHEST)
    xw_ref[...] = xw
    lsld_ref[...] = jnp.dot(xw, a_ref[...], preferred_element_type=jnp.float32,
                 precision=lax.Precision.HIGHEST)


def _tc_pre(x, W, A):
    din = x.shape[1]
    return pl.pallas_call(
        _pre_body,
        grid=(x.shape[0] // ROWS,),
        in_specs=[
            pl.BlockSpec((ROWS, din), lambda i: (i, 0)),
            pl.BlockSpec((din, 128), lambda i: (0, 0)),
            pl.BlockSpec((128, LSW), lambda i: (0, 0)),
        ],
        out_specs=[
            pl.BlockSpec((ROWS, 128), lambda i: (i, 0)),
            pl.BlockSpec((ROWS, LSW), lambda i: (i, 0)),
        ],
        out_shape=[
            jax.ShapeDtypeStruct((x.shape[0], 128), jnp.float32),
            jax.ShapeDtypeStruct((x.shape[0], LSW), jnp.float32),
        ],
    )(x, W, A)


def _combine(accf, accd, xw, l):
    w0 = jnp.exp(_leaky(l[:, 0:1] + l[:, 2:3]))
    w1 = jnp.exp(_leaky(l[:, 1:2] + l[:, 3:4]))
    den0 = accd[:, 0:1] + w0 + 1e-16
    den1 = accd[:, 1:2] + w1 + 1e-16
    r = accf.shape[0]
    wb = jnp.concatenate(
        [jnp.broadcast_to(w0, (r, 64)), jnp.broadcast_to(w1, (r, 64))], axis=1)
    denb = jnp.concatenate(
        [jnp.broadcast_to(den0, (r, 64)), jnp.broadcast_to(den1, (r, 64))], axis=1)
    return (accf + wb * xw) / denb


def _mid_body(accf_ref, accd_ref, xw_ref, lsld_ref, w2_ref, a2_ref, b1_ref,
              xw2_ref, lsld2_ref):
    x1 = _combine(accf_ref[...], accd_ref[...], xw_ref[...],
                  lsld_ref[...]) + b1_ref[...]
    xw2 = jnp.dot(x1, w2_ref[...], preferred_element_type=jnp.float32,
                 precision=lax.Precision.HIGHEST)
    xw2_ref[...] = xw2
    lsld2_ref[...] = jnp.dot(xw2, a2_ref[...], preferred_element_type=jnp.float32,
                 precision=lax.Precision.HIGHEST)


def _tc_mid(accf, accd, xw, lsld, W2, A2, b1):
    n = xw.shape[0]
    return pl.pallas_call(
        _mid_body,
        grid=(n // ROWS,),
        in_specs=[
            pl.BlockSpec((ROWS, 128), lambda i: (i, 0)),
            pl.BlockSpec((ROWS, 16), lambda i: (i, 0)),
            pl.BlockSpec((ROWS, 128), lambda i: (i, 0)),
            pl.BlockSpec((ROWS, LSW), lambda i: (i, 0)),
            pl.BlockSpec((128, 128), lambda i: (0, 0)),
            pl.BlockSpec((128, LSW), lambda i: (0, 0)),
            pl.BlockSpec((1, 128), lambda i: (0, 0)),
        ],
        out_specs=[
            pl.BlockSpec((ROWS, 128), lambda i: (i, 0)),
            pl.BlockSpec((ROWS, LSW), lambda i: (i, 0)),
        ],
        out_shape=[
            jax.ShapeDtypeStruct((n, 128), jnp.float32),
            jax.ShapeDtypeStruct((n, LSW), jnp.float32),
        ],
    )(accf, accd, xw, lsld, W2, A2, b1)


def _post_body(accf_ref, accd_ref, xw_ref, lsld_ref, b2_ref, wfc_ref, sc_ref,
               sh_ref, y_ref):
    x2 = _combine(accf_ref[...], accd_ref[...], xw_ref[...],
                  lsld_ref[...]) + b2_ref[...]
    t = jnp.dot(x2, wfc_ref[...], preferred_element_type=jnp.float32,
                 precision=lax.Precision.HIGHEST)
    t = t * sc_ref[...] + sh_ref[...]
    y_ref[...] = jnp.maximum(t, 0.0)


def _tc_post(accf, accd, xw, lsld, b2, Wfc, scale, shift):
    n = xw.shape[0]
    return pl.pallas_call(
        _post_body,
        grid=(n // ROWS,),
        in_specs=[
            pl.BlockSpec((ROWS, 128), lambda i: (i, 0)),
            pl.BlockSpec((ROWS, 16), lambda i: (i, 0)),
            pl.BlockSpec((ROWS, 128), lambda i: (i, 0)),
            pl.BlockSpec((ROWS, LSW), lambda i: (i, 0)),
            pl.BlockSpec((1, 128), lambda i: (0, 0)),
            pl.BlockSpec((128, 1), lambda i: (0, 0)),
            pl.BlockSpec((1, 1), lambda i: (0, 0)),
            pl.BlockSpec((1, 1), lambda i: (0, 0)),
        ],
        out_specs=pl.BlockSpec((ROWS, 1), lambda i: (i, 0)),
        out_shape=jax.ShapeDtypeStruct((n, 1), jnp.float32),
    )(accf, accd, xw, lsld, b2, Wfc, scale, shift)


SCH = 25
RPT = NN // NT


def _sc_body(src_hbm, dst_hbm, lsld_hbm, xw_hbm, outF_hbm, outD_hbm,
             slab_s, slab_d, idxs_v, idxd_v, lsS_v, lsD_v, w_v, g_v, den_v,
             z_v, zd_v, accF, accD, lsem, gsem0, gsem1, ssem0, ssem1):
    c_ax = lax.axis_index("c")
    s_ax = lax.axis_index("s")
    row0 = s_ax * RPT
    iota = lax.iota(jnp.int32, 16)
    gsems = (gsem0, gsem1)
    ssems = (ssem0, ssem1)

    def _zz(i, _):
        for q in range(8):
            z_v[i, pl.ds(q * 16, 16)] = jnp.zeros((16,), jnp.float32)
        zd_v[i, pl.ds(0, 16)] = jnp.zeros((16,), jnp.float32)
        return 0
    lax.fori_loop(0, 25, _zz, 0)

    def _zero_acc(k, _):
        pltpu.sync_copy(z_v, accF.at[pl.ds(row0 + k * 25, 25)])
        pltpu.sync_copy(zd_v, accD.at[pl.ds(row0 + k * 25, 25)])
        return 0

    lax.fori_loop(0, RPT // 25, _zero_acc, 0)

    def _stage_slab(sup):
        slot = jnp.remainder(jnp.int32(sup), 2)
        pltpu.sync_copy(src_hbm.at[s_ax, pl.ds(sup * SCH, SCH)],
                        slab_s.at[slot])
        pltpu.sync_copy(dst_hbm.at[s_ax, pl.ds(sup * SCH, SCH)],
                        slab_d.at[slot])

    def _prep_idx(n, ns, off):
        sup = jnp.int32(n) // SCH
        loc = jnp.remainder(jnp.int32(n), SCH)
        slot = jnp.remainder(sup, 2)
        for j in range(CH // 16):
            sl = pl.ds(j * 16, 16)
            idxs_v[ns, sl] = slab_s[slot, loc, sl] + off
            idxd_v[ns, sl] = slab_d[slot, loc, sl] + off

    def _fire(ns):
        pltpu.async_copy(lsld_hbm.at[idxs_v.at[ns]], lsS_v, lsem)
        pltpu.async_copy(lsld_hbm.at[idxd_v.at[ns]], lsD_v, lsem)
        pltpu.async_copy(xw_hbm.at[idxs_v.at[ns]], g_v.at[ns], gsems[ns])

    def _drain_ls():
        pltpu.make_async_copy(lsld_hbm.at[pl.ds(0, CH)], lsS_v, lsem).wait()
        pltpu.make_async_copy(lsld_hbm.at[pl.ds(0, CH)], lsD_v, lsem).wait()

    def _drain_g(ns):
        pltpu.make_async_copy(xw_hbm.at[pl.ds(0, CH)], g_v.at[ns],
                              gsems[ns]).wait()

    def _drain_s(ns):
        pltpu.make_async_copy(xw_hbm.at[pl.ds(0, CH)], g_v.at[ns],
                              ssems[ns]).wait()
        pltpu.make_async_copy(lsld_hbm.at[pl.ds(0, CH)], den_v.at[ns],
                              ssems[ns]).wait()

    def _process(cc, ns, off, do_next):
        nn_ = 1 - ns
        _drain_ls()
        for jj in range(CH // 16):
            ridx = iota + jj * 16
            ls0 = plsc.load_gather(lsS_v, [ridx, iota * 0])
            ls1 = plsc.load_gather(lsS_v, [ridx, iota * 0 + 1])
            ld0 = plsc.load_gather(lsD_v, [ridx, iota * 0 + 2])
            ld1 = plsc.load_gather(lsD_v, [ridx, iota * 0 + 3])
            w_v[0, pl.ds(jj * 16, 16)] = jnp.exp(_leaky(ls0 + ld0))
            w_v[1, pl.ds(jj * 16, 16)] = jnp.exp(_leaky(ls1 + ld1))
        if do_next:
            n = cc + 1
            loc_n = jnp.remainder(jnp.int32(n), SCH)

            @pl.when(loc_n == 0)
            def _():
                _stage_slab(jnp.int32(n) // SCH)

            _prep_idx(n, nn_, off)

            @pl.when(jnp.int32(cc) >= 1)
            def _():
                _drain_s(nn_)

            _fire(nn_)

        _drain_g(ns)

        def _scale(jj, _2):
            base = jj * 16
            w0v = w_v[0, pl.ds(base, 16)]
            w1v = w_v[1, pl.ds(base, 16)]
            for l in range(16):
                j = base + l
                w0 = w0v[l]
                w1 = w1v[l]
                for q in range(4):
                    sl = pl.ds(q * 16, 16)
                    g_v[ns, j, sl] = g_v[ns, j, sl] * w0
                for q in range(4, 8):
                    sl = pl.ds(q * 16, 16)
                    g_v[ns, j, sl] = g_v[ns, j, sl] * w1
                den_v[ns, j, pl.ds(0, 16)] = jnp.where(
                    iota == 0, w0, jnp.where(iota == 1, w1, 0.0))
            return 0

        lax.fori_loop(0, CH // 16, _scale, 0)

        sup_c = jnp.int32(cc) // SCH
        drow = slab_d.at[jnp.remainder(sup_c, 2), jnp.remainder(jnp.int32(cc), SCH)]
        pltpu.async_copy(g_v.at[ns], accF.at[drow], ssems[ns], add=True)
        pltpu.async_copy(den_v.at[ns], accD.at[drow], ssems[ns], add=True)

    for r in range(BB // NC):
        b = 2 * r + c_ax
        off = b * NN
        plsc.subcore_barrier()

        _stage_slab(0)
        _prep_idx(0, 0, off)
        _fire(0)

        def _pair(p, _):
            _process(2 * p, 0, off, True)
            _process(2 * p + 1, 1, off, True)
            return 0

        lax.fori_loop(0, (NCHUNK - 1) // 2, _pair, 0)
        _process(NCHUNK - 1, 0, off, False)
        _drain_s(1)
        _drain_s(0)

        plsc.subcore_barrier()

        pltpu.sync_copy(accF.at[pl.ds(row0, RPT)],
                        outF_hbm.at[b, pl.ds(row0, RPT)])
        pltpu.sync_copy(accD.at[pl.ds(row0, RPT)],
                        outD_hbm.at[b, pl.ds(row0, RPT)])

        if r + 1 < BB // NC:
            lax.fori_loop(0, RPT // 25, _zero_acc, 0)


def _sc_edge(src_t, dst_t, lsld, xw):
    mesh = plsc.VectorSubcoreMesh(core_axis_name="c", subcore_axis_name="s")
    f = pl.kernel(
        _sc_body,
        mesh=mesh,
        compiler_params=pltpu.CompilerParams(
            use_tc_tiling_on_sc=False, needs_layout_passes=False),
        out_type=[
            jax.ShapeDtypeStruct((BB, NN, 128), jnp.float32),
            jax.ShapeDtypeStruct((BB, NN, 16), jnp.float32),
        ],
        scratch_types=[
            pltpu.VMEM((2, SCH, CH), jnp.int32),
            pltpu.VMEM((2, SCH, CH), jnp.int32),
            pltpu.VMEM((2, CH), jnp.int32),
            pltpu.VMEM((2, CH), jnp.int32),
            pltpu.VMEM((CH, LSW), jnp.float32),
            pltpu.VMEM((CH, LSW), jnp.float32),
            pltpu.VMEM((2, CH), jnp.float32),
            pltpu.VMEM((2, CH, 128), jnp.float32),
            pltpu.VMEM((2, CH, 16), jnp.float32),
            pltpu.VMEM((25, 128), jnp.float32),
            pltpu.VMEM((25, 16), jnp.float32),
            pltpu.VMEM_SHARED((NN, 128), jnp.float32),
            pltpu.VMEM_SHARED((NN, 16), jnp.float32),
            pltpu.SemaphoreType.DMA,
            pltpu.SemaphoreType.DMA,
            pltpu.SemaphoreType.DMA,
            pltpu.SemaphoreType.DMA,
            pltpu.SemaphoreType.DMA,
        ],
    )
    return f(src_t, dst_t, lsld, xw)


def _build_A(a_src, a_dst):
    z = jnp.zeros((64,), jnp.float32)
    top = jnp.stack([a_src[0], z, a_dst[0], z] + [z] * 12, axis=1)
    bot = jnp.stack([z, a_src[1], z, a_dst[1]] + [z] * 12, axis=1)
    return jnp.concatenate([top, bot], axis=0)


def kernel(data, org_edge_index, edge_index_set0, W1, att_src1, att_dst1,
           bias1, W2, att_src2, att_dst2, bias2, Wfc, bfc, bn_gamma, bn_beta):
    x = data.reshape(BB * NN, -1)
    src_t = edge_index_set0[0].astype(jnp.int32).reshape(NT, NCHUNK, CH)
    dst_t = edge_index_set0[1].astype(jnp.int32).reshape(NT, NCHUNK, CH)
    A1 = _build_A(att_src1, att_dst1)
    A2 = _build_A(att_src2, att_dst2)

    xw1, lsld1 = _tc_pre(x, W1, A1)
    acc1f, acc1d = _sc_edge(src_t, dst_t, lsld1, xw1)
    xw2, lsld2 = _tc_mid(acc1f.reshape(BB * NN, 128),
                         acc1d.reshape(BB * NN, 16), xw1, lsld1, W2, A2,
                         bias1.reshape(1, 128))
    acc2f, acc2d = _sc_edge(src_t, dst_t, lsld2, xw2)

    k = 1.0 / np.sqrt(1.0 + 1e-5)
    scale = (bn_gamma * k).reshape(1, 1)
    shift = (bfc * bn_gamma * k + bn_beta).reshape(1, 1)
    y = _tc_post(acc2f.reshape(BB * NN, 128), acc2d.reshape(BB * NN, 16),
                 xw2, lsld2, bias2.reshape(1, 128), Wfc, scale, shift)
    return y.reshape(BB, 1, NN)


# speedup vs baseline: 179.3375x; 1.1481x over previous
import functools

import jax
import jax.numpy as jnp
import numpy as np
from jax import lax
from jax.experimental import pallas as pl
from jax.experimental.pallas import tpu as pltpu
from jax.experimental.pallas import tpu_sc as plsc

NN = 10000
BB = 4
EE = 160000
NT = 16
NC = 2
EPT = EE // NT
CH = 80
NCHUNK = EPT // CH
ROWS = 2000
ACCW = 144
LSW = 16


def _leaky(v):
    return jnp.where(v >= 0.0, v, 0.2 * v)


def _lsld(xw, asrc, adst):
    ps = xw * asrc
    pd = xw * adst
    ls0 = jnp.sum(ps[:, 0:64], axis=1, keepdims=True)
    ls1 = jnp.sum(ps[:, 64:128], axis=1, keepdims=True)
    ld0 = jnp.sum(pd[:, 0:64], axis=1, keepdims=True)
    ld1 = jnp.sum(pd[:, 64:128], axis=1, keepdims=True)
    z = jnp.zeros((xw.shape[0], LSW - 4), jnp.float32)
    return jnp.concatenate([ls0, ls1, ld0, ld1, z], axis=1)


def _pre_body(x_ref, w_ref, asrc_ref, adst_ref, xw_ref, lsld_ref):
    xw = jnp.dot(x_ref[...], w_ref[...], preferred_element_type=jnp.float32)
    xw_ref[...] = xw
    lsld_ref[...] = _lsld(xw, asrc_ref[...], adst_ref[...])


def _tc_pre(x, W, asrc, adst):
    din = x.shape[1]
    return pl.pallas_call(
        _pre_body,
        grid=(x.shape[0] // ROWS,),
        in_specs=[
            pl.BlockSpec((ROWS, din), lambda i: (i, 0)),
            pl.BlockSpec((din, 128), lambda i: (0, 0)),
            pl.BlockSpec((1, 128), lambda i: (0, 0)),
            pl.BlockSpec((1, 128), lambda i: (0, 0)),
        ],
        out_specs=[
            pl.BlockSpec((ROWS, 128), lambda i: (i, 0)),
            pl.BlockSpec((ROWS, LSW), lambda i: (i, 0)),
        ],
        out_shape=[
            jax.ShapeDtypeStruct((x.shape[0], 128), jnp.float32),
            jax.ShapeDtypeStruct((x.shape[0], LSW), jnp.float32),
        ],
    )(x, W, asrc, adst)


def _combine(accf, accd, xw, l):
    w0 = jnp.exp(_leaky(l[:, 0:1] + l[:, 2:3]))
    w1 = jnp.exp(_leaky(l[:, 1:2] + l[:, 3:4]))
    den0 = accd[:, 0:1] + w0 + 1e-16
    den1 = accd[:, 1:2] + w1 + 1e-16
    r = accf.shape[0]
    wb = jnp.concatenate(
        [jnp.broadcast_to(w0, (r, 64)), jnp.broadcast_to(w1, (r, 64))], axis=1)
    denb = jnp.concatenate(
        [jnp.broadcast_to(den0, (r, 64)), jnp.broadcast_to(den1, (r, 64))], axis=1)
    return (accf + wb * xw) / denb


def _mid_body(accf_ref, accd_ref, xw_ref, lsld_ref, w2_ref, asrc_ref,
              adst_ref, b1_ref, xw2_ref, lsld2_ref):
    x1 = _combine(accf_ref[...], accd_ref[...], xw_ref[...],
                  lsld_ref[...]) + b1_ref[...]
    xw2 = jnp.dot(x1, w2_ref[...], preferred_element_type=jnp.float32)
    xw2_ref[...] = xw2
    lsld2_ref[...] = _lsld(xw2, asrc_ref[...], adst_ref[...])


def _tc_mid(accf, accd, xw, lsld, W2, asrc, adst, b1):
    n = xw.shape[0]
    return pl.pallas_call(
        _mid_body,
        grid=(n // ROWS,),
        in_specs=[
            pl.BlockSpec((ROWS, 128), lambda i: (i, 0)),
            pl.BlockSpec((ROWS, 16), lambda i: (i, 0)),
            pl.BlockSpec((ROWS, 128), lambda i: (i, 0)),
            pl.BlockSpec((ROWS, LSW), lambda i: (i, 0)),
            pl.BlockSpec((128, 128), lambda i: (0, 0)),
            pl.BlockSpec((1, 128), lambda i: (0, 0)),
            pl.BlockSpec((1, 128), lambda i: (0, 0)),
            pl.BlockSpec((1, 128), lambda i: (0, 0)),
        ],
        out_specs=[
            pl.BlockSpec((ROWS, 128), lambda i: (i, 0)),
            pl.BlockSpec((ROWS, LSW), lambda i: (i, 0)),
        ],
        out_shape=[
            jax.ShapeDtypeStruct((n, 128), jnp.float32),
            jax.ShapeDtypeStruct((n, LSW), jnp.float32),
        ],
    )(accf, accd, xw, lsld, W2, asrc, adst, b1)


def _post_body(accf_ref, accd_ref, xw_ref, lsld_ref, b2_ref, wfc_ref, sc_ref,
               sh_ref, y_ref):
    x2 = _combine(accf_ref[...], accd_ref[...], xw_ref[...],
                  lsld_ref[...]) + b2_ref[...]
    t = jnp.dot(x2, wfc_ref[...], preferred_element_type=jnp.float32)
    t = t * sc_ref[...] + sh_ref[...]
    y_ref[...] = jnp.maximum(t, 0.0)


def _tc_post(accf, accd, xw, lsld, b2, Wfc, scale, shift):
    n = xw.shape[0]
    return pl.pallas_call(
        _post_body,
        grid=(n // ROWS,),
        in_specs=[
            pl.BlockSpec((ROWS, 128), lambda i: (i, 0)),
            pl.BlockSpec((ROWS, 16), lambda i: (i, 0)),
            pl.BlockSpec((ROWS, 128), lambda i: (i, 0)),
            pl.BlockSpec((ROWS, LSW), lambda i: (i, 0)),
            pl.BlockSpec((1, 128), lambda i: (0, 0)),
            pl.BlockSpec((128, 1), lambda i: (0, 0)),
            pl.BlockSpec((1, 1), lambda i: (0, 0)),
            pl.BlockSpec((1, 1), lambda i: (0, 0)),
        ],
        out_specs=pl.BlockSpec((ROWS, 1), lambda i: (i, 0)),
        out_shape=jax.ShapeDtypeStruct((n, 1), jnp.float32),
    )(accf, accd, xw, lsld, b2, Wfc, scale, shift)


SCH = 25
RPT = NN // NT


def _sc_body(src_hbm, dst_hbm, lsld_hbm, xw_hbm, outF_hbm, outD_hbm,
             slab_s, slab_d, idxs_v, idxd_v, lsS_v, lsD_v, g_v, den_v,
             z_v, zd_v, accF, accD, lsem0, lsem1, gsem0, gsem1, ssem0, ssem1):
    c_ax = lax.axis_index("c")
    s_ax = lax.axis_index("s")
    row0 = s_ax * RPT
    iota = lax.iota(jnp.int32, 16)
    lsems = (lsem0, lsem1)
    gsems = (gsem0, gsem1)
    ssems = (ssem0, ssem1)

    def _zz(i, _):
        for q in range(8):
            z_v[i, pl.ds(q * 16, 16)] = jnp.zeros((16,), jnp.float32)
        zd_v[i, pl.ds(0, 16)] = jnp.zeros((16,), jnp.float32)
        return 0
    lax.fori_loop(0, 25, _zz, 0)

    def _zden(j, _):
        den_v[0, j, pl.ds(0, 16)] = jnp.zeros((16,), jnp.float32)
        den_v[1, j, pl.ds(0, 16)] = jnp.zeros((16,), jnp.float32)
        return 0
    lax.fori_loop(0, CH, _zden, 0)

    def _zero_acc(k, _):
        pltpu.sync_copy(z_v, accF.at[pl.ds(row0 + k * 25, 25)])
        pltpu.sync_copy(zd_v, accD.at[pl.ds(row0 + k * 25, 25)])
        return 0

    lax.fori_loop(0, RPT // 25, _zero_acc, 0)

    def _stage_slab(sup):
        slot = jnp.remainder(jnp.int32(sup), 2)
        pltpu.sync_copy(src_hbm.at[s_ax, pl.ds(sup * SCH, SCH)],
                        slab_s.at[slot])
        pltpu.sync_copy(dst_hbm.at[s_ax, pl.ds(sup * SCH, SCH)],
                        slab_d.at[slot])

    def _prep_idx(n, ns, off):
        sup = jnp.int32(n) // SCH
        loc = jnp.remainder(jnp.int32(n), SCH)
        slot = jnp.remainder(sup, 2)
        for j in range(CH // 16):
            sl = pl.ds(j * 16, 16)
            idxs_v[ns, sl] = slab_s[slot, loc, sl] + off
            idxd_v[ns, sl] = slab_d[slot, loc, sl] + off

    def _fire(ns):
        pltpu.async_copy(lsld_hbm.at[idxs_v.at[ns]], lsS_v.at[ns], lsems[ns])
        pltpu.async_copy(lsld_hbm.at[idxd_v.at[ns]], lsD_v.at[ns], lsems[ns])
        pltpu.async_copy(xw_hbm.at[idxs_v.at[ns]], g_v.at[ns], gsems[ns])

    def _drain_ls(ns):
        pltpu.make_async_copy(lsld_hbm.at[pl.ds(0, CH)], lsS_v.at[ns],
                              lsems[ns]).wait()
        pltpu.make_async_copy(lsld_hbm.at[pl.ds(0, CH)], lsD_v.at[ns],
                              lsems[ns]).wait()

    def _drain_g(ns):
        pltpu.make_async_copy(xw_hbm.at[pl.ds(0, CH)], g_v.at[ns],
                              gsems[ns]).wait()

    def _drain_s(ns):
        pltpu.make_async_copy(xw_hbm.at[pl.ds(0, CH)], g_v.at[ns],
                              ssems[ns]).wait()
        pltpu.make_async_copy(lsld_hbm.at[pl.ds(0, CH)], den_v.at[ns],
                              ssems[ns]).wait()

    def _process(cc, ns, off, do_next):
        nn_ = 1 - ns
        if do_next:
            n = cc + 1
            loc_n = jnp.remainder(jnp.int32(n), SCH)

            @pl.when(loc_n == 0)
            def _():
                _stage_slab(jnp.int32(n) // SCH)

            _prep_idx(n, nn_, off)

            @pl.when(jnp.int32(cc) >= 1)
            def _():
                _drain_s(nn_)

            _fire(nn_)

        _drain_ls(ns)
        _drain_g(ns)
        nsv = jnp.full((16,), ns, jnp.int32)

        def _scale(jj, _2):
            ridx = iota + jj * 16
            ls0 = plsc.load_gather(lsS_v, [nsv, ridx, iota * 0])
            ls1 = plsc.load_gather(lsS_v, [nsv, ridx, iota * 0 + 1])
            ld0 = plsc.load_gather(lsD_v, [nsv, ridx, iota * 0 + 2])
            ld1 = plsc.load_gather(lsD_v, [nsv, ridx, iota * 0 + 3])
            w0v = jnp.exp(_leaky(ls0 + ld0))
            w1v = jnp.exp(_leaky(ls1 + ld1))
            plsc.store_scatter(den_v, [nsv, ridx, iota * 0], w0v)
            plsc.store_scatter(den_v, [nsv, ridx, iota * 0 + 1], w1v)
            base = jj * 16
            for l in range(16):
                j = base + l
                w0 = w0v[l]
                w1 = w1v[l]
                for q in range(4):
                    sl = pl.ds(q * 16, 16)
                    g_v[ns, j, sl] = g_v[ns, j, sl] * w0
                for q in range(4, 8):
                    sl = pl.ds(q * 16, 16)
                    g_v[ns, j, sl] = g_v[ns, j, sl] * w1
            return 0

        lax.fori_loop(0, CH // 16, _scale, 0)

        sup_c = jnp.int32(cc) // SCH
        drow = slab_d.at[jnp.remainder(sup_c, 2), jnp.remainder(jnp.int32(cc), SCH)]
        pltpu.async_copy(g_v.at[ns], accF.at[drow], ssems[ns], add=True)
        pltpu.async_copy(den_v.at[ns], accD.at[drow], ssems[ns], add=True)

    for r in range(BB // NC):
        b = 2 * r + c_ax
        off = b * NN
        plsc.subcore_barrier()

        _stage_slab(0)
        _prep_idx(0, 0, off)
        _fire(0)

        def _pair(p, _):
            _process(2 * p, 0, off, True)
            _process(2 * p + 1, 1, off, True)
            return 0

        lax.fori_loop(0, (NCHUNK - 1) // 2, _pair, 0)
        _process(NCHUNK - 1, 0, off, False)
        _drain_s(1)
        _drain_s(0)

        plsc.subcore_barrier()

        pltpu.sync_copy(accF.at[pl.ds(row0, RPT)],
                        outF_hbm.at[b, pl.ds(row0, RPT)])
        pltpu.sync_copy(accD.at[pl.ds(row0, RPT)],
                        outD_hbm.at[b, pl.ds(row0, RPT)])

        if r + 1 < BB // NC:
            lax.fori_loop(0, RPT // 25, _zero_acc, 0)


def _sc_edge(src_t, dst_t, lsld, xw):
    mesh = plsc.VectorSubcoreMesh(core_axis_name="c", subcore_axis_name="s")
    f = pl.kernel(
        _sc_body,
        mesh=mesh,
        compiler_params=pltpu.CompilerParams(
            use_tc_tiling_on_sc=False, needs_layout_passes=False),
        out_type=[
            jax.ShapeDtypeStruct((BB, NN, 128), jnp.float32),
            jax.ShapeDtypeStruct((BB, NN, 16), jnp.float32),
        ],
        scratch_types=[
            pltpu.VMEM((2, SCH, CH), jnp.int32),
            pltpu.VMEM((2, SCH, CH), jnp.int32),
            pltpu.VMEM((2, CH), jnp.int32),
            pltpu.VMEM((2, CH), jnp.int32),
            pltpu.VMEM((2, CH, LSW), jnp.float32),
            pltpu.VMEM((2, CH, LSW), jnp.float32),
            pltpu.VMEM((2, CH, 128), jnp.float32),
            pltpu.VMEM((2, CH, 16), jnp.float32),
            pltpu.VMEM((25, 128), jnp.float32),
            pltpu.VMEM((25, 16), jnp.float32),
            pltpu.VMEM_SHARED((NN, 128), jnp.float32),
            pltpu.VMEM_SHARED((NN, 16), jnp.float32),
            pltpu.SemaphoreType.DMA,
            pltpu.SemaphoreType.DMA,
            pltpu.SemaphoreType.DMA,
            pltpu.SemaphoreType.DMA,
            pltpu.SemaphoreType.DMA,
            pltpu.SemaphoreType.DMA,
        ],
    )
    return f(src_t, dst_t, lsld, xw)


def kernel(data, org_edge_index, edge_index_set0, W1, att_src1, att_dst1,
           bias1, W2, att_src2, att_dst2, bias2, Wfc, bfc, bn_gamma, bn_beta):
    x = data.reshape(BB * NN, -1)
    src_t = edge_index_set0[0].astype(jnp.int32).reshape(NT, NCHUNK, CH)
    dst_t = edge_index_set0[1].astype(jnp.int32).reshape(NT, NCHUNK, CH)
    xw1, lsld1 = _tc_pre(x, W1, att_src1.reshape(1, 128),
                         att_dst1.reshape(1, 128))
    acc1f, acc1d = _sc_edge(src_t, dst_t, lsld1, xw1)
    xw2, lsld2 = _tc_mid(acc1f.reshape(BB * NN, 128),
                         acc1d.reshape(BB * NN, 16), xw1, lsld1, W2,
                         att_src2.reshape(1, 128), att_dst2.reshape(1, 128),
                         bias1.reshape(1, 128))
    acc2f, acc2d = _sc_edge(src_t, dst_t, lsld2, xw2)

    k = 1.0 / np.sqrt(1.0 + 1e-5)
    scale = (bn_gamma * k).reshape(1, 1)
    shift = (bfc * bn_gamma * k + bn_beta).reshape(1, 1)
    y = _tc_post(acc2f.reshape(BB * NN, 128), acc2d.reshape(BB * NN, 16),
                 xw2, lsld2, bias2.reshape(1, 128), Wfc, scale, shift)
    return y.reshape(BB, 1, NN)
